# Initial kernel scaffold; baseline (speedup 1.0000x reference)
import jax
import jax.numpy as jnp
from jax.experimental import pallas as pl


def kernel(Z, R, idx_i, idx_j, M, QaAlpha, QaBeta, batch_seg, params):
    raise NotImplementedError("write your pallas kernel here")


# speedup vs baseline: 2.1390x; 2.1390x over previous
import functools

import numpy as np
import jax
import jax.numpy as jnp
from jax import lax
from jax.experimental import pallas as pl
from jax.experimental.pallas import tpu as pltpu
from jax.experimental.pallas import tpu_sc as plsc

F = 128
K = 64
NBLK = 5
NOUT = 2
NRI = 3
NRA = 2
NRO = 1
SR_CUT = 8.0

_NC = 2
_NS = 16
_NW = _NC * _NS
_L = 16

_HI = lax.Precision.HIGHEST
_LOG2 = np.float32(np.log(2.0))


def _act(x):
    return jnp.logaddexp(x, 0.0) - _LOG2


def _mm(a, w):
    return jnp.dot(a, w, preferred_element_type=jnp.float32, precision=_HI)


def _sc_d2(rx, ry, rz, ii, jj):
    n = rx.shape[0]
    e = ii.shape[0]
    ew = e // _NW
    mesh = plsc.VectorSubcoreMesh(core_axis_name="c", subcore_axis_name="s")

    @functools.partial(
        pl.kernel,
        out_type=jax.ShapeDtypeStruct((e,), jnp.float32),
        mesh=mesh,
        compiler_params=pltpu.CompilerParams(needs_layout_passes=False),
        scratch_types=[
            pltpu.VMEM((n,), jnp.float32),
            pltpu.VMEM((n,), jnp.float32),
            pltpu.VMEM((n,), jnp.float32),
            pltpu.VMEM((ew,), jnp.int32),
            pltpu.VMEM((ew,), jnp.int32),
            pltpu.VMEM((ew,), jnp.float32),
        ],
    )
    def k(rx_h, ry_h, rz_h, ii_h, jj_h, d2_h, rxv, ryv, rzv, iiv, jjv, d2v):
        wid = lax.axis_index("s") * _NC + lax.axis_index("c")
        base = wid * ew
        pltpu.sync_copy(rx_h, rxv)
        pltpu.sync_copy(ry_h, ryv)
        pltpu.sync_copy(rz_h, rzv)
        pltpu.sync_copy(ii_h.at[pl.ds(base, ew)], iiv)
        pltpu.sync_copy(jj_h.at[pl.ds(base, ew)], jjv)

        def body(t, carry):
            sl = pl.ds(t * _L, _L)
            ai = iiv[sl]
            aj = jjv[sl]
            dx = plsc.load_gather(rxv, [ai]) - plsc.load_gather(rxv, [aj])
            dy = plsc.load_gather(ryv, [ai]) - plsc.load_gather(ryv, [aj])
            dz = plsc.load_gather(rzv, [ai]) - plsc.load_gather(rzv, [aj])
            d2v[sl] = dx * dx + dy * dy + dz * dz
            return carry

        lax.fori_loop(0, ew // _L, body, 0)
        pltpu.sync_copy(d2v, d2_h.at[pl.ds(base, ew)])

    return k(rx, ry, rz, ii, jj)


def _sc_msg(gflat, xj, ii, jj, b):
    n = xj.shape[0]
    e = ii.shape[0]
    ew = e // _NW
    c = 80
    nchunk = ew // c
    rps = (n // _NS) // 8 * 8
    tail = n - rps * _NS
    mesh = plsc.VectorSubcoreMesh(core_axis_name="c", subcore_axis_name="s")

    @functools.partial(
        pl.kernel,
        out_type=jax.ShapeDtypeStruct((_NC, n, F), jnp.float32),
        mesh=mesh,
        compiler_params=pltpu.CompilerParams(needs_layout_passes=False),
        scratch_types=[
            pltpu.VMEM((c,), jnp.int32),
            pltpu.VMEM((c,), jnp.int32),
            pltpu.VMEM((c, F), jnp.float32),
            pltpu.VMEM((c, F), jnp.float32),
            pltpu.VMEM_SHARED((n, F), jnp.float32),
            pltpu.SemaphoreType.DMA,
        ],
    )
    def k(g_h, xj_h, ii_h, jj_h, zero_h, out_h, iiv, jjv, rowsv, gv, msh, sem):
        cid = lax.axis_index("c")
        sid = lax.axis_index("s")
        wid = sid * _NC + cid

        pltpu.sync_copy(zero_h.at[pl.ds(sid * rps, rps)],
                        msh.at[pl.ds(sid * rps, rps)])
        if tail:
            @pl.when(sid == 0)
            def _():
                pltpu.sync_copy(zero_h.at[pl.ds(rps * _NS, tail)],
                                msh.at[pl.ds(rps * _NS, tail)])
        plsc.subcore_barrier()

        ebase = wid * ew

        def chunk(kk, carry):
            base = ebase + kk * c
            pltpu.sync_copy(ii_h.at[pl.ds(base, c)], iiv)
            pltpu.sync_copy(jj_h.at[pl.ds(base, c)], jjv)
            pltpu.sync_copy(g_h.at[pl.ds(b * e + base, c)], gv)
            pltpu.async_copy(xj_h.at[jjv], rowsv, sem).wait()

            def mul(ee, cc):
                for f in range(F // _L):
                    sl = pl.ds(f * _L, _L)
                    rowsv[ee, sl] = rowsv[ee, sl] * gv[ee, sl]
                return cc

            lax.fori_loop(0, c, mul, 0)
            pltpu.sync_copy(rowsv, msh.at[iiv], add=True)
            return carry

        lax.fori_loop(0, nchunk, chunk, 0)
        plsc.subcore_barrier()

        if tail:
            @pl.when(sid == 0)
            def _():
                pltpu.sync_copy(msh.at[pl.ds(rps * _NS, tail)],
                                out_h.at[cid, pl.ds(rps * _NS, tail)])
        pltpu.sync_copy(msh.at[pl.ds(sid * rps, rps)],
                        out_h.at[cid, pl.ds(sid * rps, rps)])

    zeros = jnp.zeros((n, F), jnp.float32)
    return k(gflat, xj, ii, jj, zeros)


_T = 2000


def _tc_g(d2col, g_stack, centers):
    e = d2col.shape[0]
    nt = e // _T
    w = float(((2.0 / K) * (1.0 - np.exp(-SR_CUT))) ** (-2))

    def body(d2_ref, g_ref, c_ref, o_ref):
        d = jnp.sqrt(d2_ref[...])
        r = d * np.float32(1.0 / SR_CUT)
        r2 = r * r
        r3 = r2 * r
        r4 = r2 * r2
        r5 = r4 * r
        fcut = jnp.where(r < 1.0, 1.0 - 6.0 * r5 + 15.0 * r4 - 10.0 * r3, 0.0)
        diff = jnp.exp(-d) - c_ref[...]
        rbf = fcut * jnp.exp(np.float32(-w) * (diff * diff))
        o_ref[...] = _mm(rbf, g_ref[0])

    return pl.pallas_call(
        body,
        grid=(NBLK, nt),
        in_specs=[
            pl.BlockSpec((_T, 1), lambda b, i: (i, 0)),
            pl.BlockSpec((1, K, F), lambda b, i: (b, 0, 0)),
            pl.BlockSpec((1, K), lambda b, i: (0, 0)),
        ],
        out_specs=pl.BlockSpec((_T, F), lambda b, i: (b * nt + i, 0)),
        out_shape=jax.ShapeDtypeStruct((NBLK * e, F), jnp.float32),
    )(d2col, g_stack, centers)


def _tc_init(zf, mf, qa, qb, w0, b0, w1, b1, w2, b2, wi, bi, wj, bj):
    n = zf.shape[0]
    nt = n // _T

    def body(z_ref, m_ref, a_ref, q_ref, w0_ref, b0_ref, w1_ref, b1_ref,
             w2_ref, b2_ref, wi_ref, bi_ref, wj_ref, bj_ref,
             x_ref, xi_ref, xj_ref):
        pre = (z_ref[...] * w0_ref[0:1, :] + m_ref[...] * w0_ref[1:2, :]
               + a_ref[...] * w0_ref[2:3, :] + q_ref[...] * w0_ref[3:4, :]
               + b0_ref[...])
        x = _act(pre)
        x = _act(_mm(x, w1_ref[...]) + b1_ref[...])
        x = _mm(x, w2_ref[...]) + b2_ref[...]
        xt = _act(x)
        x_ref[...] = x
        xi_ref[...] = _act(_mm(xt, wi_ref[...]) + bi_ref[...])
        xj_ref[...] = _act(_mm(xt, wj_ref[...]) + bj_ref[...])

    col = pl.BlockSpec((_T, 1), lambda i: (i, 0))
    row = lambda r: pl.BlockSpec((r, F), lambda i: (0, 0))
    tile = pl.BlockSpec((_T, F), lambda i: (i, 0))
    return pl.pallas_call(
        body,
        grid=(nt,),
        in_specs=[col, col, col, col,
                  row(4), row(1), row(F), row(1), row(F), row(1),
                  row(F), row(1), row(F), row(1)],
        out_specs=[tile, tile, tile],
        out_shape=[jax.ShapeDtypeStruct((n, F), jnp.float32)] * 3,
    )(zf, mf, qa, qb, w0, b0, w1, b1, w2, b2, wi, bi, wj, bj)


def _tc_block(x, xi, parts, outacc, lo2, nh, wts, gate):
    n = x.shape[0]
    nt = n // _T
    inv = float(gate) / float(n * NOUT)

    def body(x_ref, xi_ref, p_ref, oa_ref, lo_ref, nh_ref,
             wri1, bri1, wri2, bri2, wm, bm, wra1, bra1, wra2, bra2,
             wro1, bro1, wro2, bro2, wout, bout, wmix, bmix,
             win, bin_, wjn, bjn,
             xo_ref, xio_ref, xjo_ref, oao_ref, loo_ref, nho_ref):
        m = xi_ref[...] + p_ref[0] + p_ref[1]
        for t in range(NRI):
            m = m + _mm(_act(_mm(_act(m), wri1[t]) + bri1[t]), wri2[t]) + bri2[t]
        x = x_ref[...] + _mm(_act(m), wm[...]) + bm[...]
        for t in range(NRA):
            x = x + _mm(_act(_mm(_act(x), wra1[t]) + bra1[t]), wra2[t]) + bra2[t]
        o = x
        for t in range(NRO):
            o = o + _mm(_act(_mm(_act(o), wro1[t]) + bro1[t]), wro2[t]) + bro2[t]
        out = _mm(_act(o), wout[...]) + bout[...]
        oao_ref[...] = oa_ref[...] + out * wmix[...] + bmix[...]
        out2 = out * out
        frac = out2 / (out2 + lo_ref[...] + np.float32(1e-7))
        loo_ref[...] = out2
        part = jnp.sum(frac) * np.float32(inv)

        @pl.when(pl.program_id(0) == 0)
        def _():
            nho_ref[...] = nh_ref[...] + part

        @pl.when(pl.program_id(0) != 0)
        def _():
            nho_ref[...] = nho_ref[...] + part

        xo_ref[...] = x
        xt = _act(x)
        xio_ref[...] = _act(_mm(xt, win[...]) + bin_[...])
        xjo_ref[...] = _act(_mm(xt, wjn[...]) + bjn[...])

    tile = pl.BlockSpec((_T, F), lambda i: (i, 0))
    scal = pl.BlockSpec((1, 1), lambda i: (0, 0))

    def wspec(a):
        nd = a.ndim
        return pl.BlockSpec(a.shape, lambda i, _nd=nd: (0,) * _nd)

    (wri1, bri1, wri2, bri2, wm, bm, wra1, bra1, wra2, bra2,
     wro1, bro1, wro2, bro2, wout, bout, wmix, bmix,
     win, bin_, wjn, bjn) = wts
    pspec = pl.BlockSpec((_NC, _T, F), lambda i: (0, i, 0))
    return pl.pallas_call(
        body,
        grid=(nt,),
        in_specs=[tile, tile, pspec, tile, tile, scal] + [wspec(a) for a in wts],
        out_specs=[tile, tile, tile, tile, tile, scal],
        out_shape=[jax.ShapeDtypeStruct((n, F), jnp.float32)] * 5
        + [jax.ShapeDtypeStruct((1, 1), jnp.float32)],
    )(x, xi, parts, outacc, lo2, nh, wri1, bri1, wri2, bri2, wm, bm,
      wra1, bra1, wra2, bra2, wro1, bro1, wro2, bro2, wout, bout,
      wmix, bmix, win, bin_, wjn, bjn)


def _tc_final(outp, zcol, scp, shp):
    n = outp.shape[0]
    nt = n // _T

    def body(o_ref, z_ref, sc_ref, sh_ref, r_ref):
        ids = lax.broadcasted_iota(jnp.int32, (_T, F), 1)
        oh = (ids == z_ref[...]).astype(jnp.float32)
        sc = _mm(oh, sc_ref[...])
        sh = _mm(oh, sh_ref[...])
        r_ref[...] = o_ref[...] * sc + sh

    tile = pl.BlockSpec((_T, F), lambda i: (i, 0))
    col = pl.BlockSpec((_T, 1), lambda i: (i, 0))
    full = pl.BlockSpec((F, F), lambda i: (0, 0))
    return pl.pallas_call(
        body,
        grid=(nt,),
        in_specs=[tile, col, full, full],
        out_specs=tile,
        out_shape=jax.ShapeDtypeStruct((n, F), jnp.float32),
    )(outp, zcol, scp, shp)


def kernel(Z, R, idx_i, idx_j, M, QaAlpha, QaBeta, batch_seg, params):
    n = Z.shape[0]
    e = idx_i.shape[0]
    p = params

    ii = idx_i.astype(jnp.int32)
    jj = idx_j.astype(jnp.int32)
    rx = R[:, 0]
    ry = R[:, 1]
    rz = R[:, 2]

    d2 = _sc_d2(rx, ry, rz, ii, jj)

    centers = jnp.linspace(
        np.float32(np.exp(-SR_CUT)), 1.0, K).astype(jnp.float32).reshape(1, K)
    gflat = _tc_g(d2.reshape(e, 1), p['G'], centers)

    zf = Z.astype(jnp.float32).reshape(n, 1)
    x, xi, xj = _tc_init(
        zf, M.reshape(n, 1), QaAlpha.reshape(n, 1), QaBeta.reshape(n, 1),
        p['Wem0'], p['bem0'].reshape(1, F),
        p['Wem1'], p['bem1'].reshape(1, F),
        p['Wem2'], p['bem2'].reshape(1, F),
        p['Wi'][0], p['bi'][0].reshape(1, F),
        p['Wj'][0], p['bj'][0].reshape(1, F))

    woutp = jnp.zeros((NBLK, F, F), jnp.float32).at[:, :, :NOUT].set(p['Wout'])
    boutp = jnp.zeros((NBLK, 1, F), jnp.float32).at[:, 0, :NOUT].set(p['bout'])
    wmixp = jnp.zeros((NBLK, 1, F), jnp.float32).at[:, 0, :NOUT].set(p['Wmix'])
    bmixp = jnp.zeros((1, F), jnp.float32).at[0, :NOUT].set(p['bmix'])

    outacc = jnp.zeros((n, F), jnp.float32)
    lo2 = jnp.zeros((n, F), jnp.float32)
    nh = jnp.zeros((1, 1), jnp.float32)

    for b in range(NBLK):
        parts = _sc_msg(gflat, xj, ii, jj, b)
        nb = (b + 1) % NBLK
        wts = (
            p['Wri1'][b], p['bri1'][b].reshape(NRI, 1, F),
            p['Wri2'][b], p['bri2'][b].reshape(NRI, 1, F),
            p['Wm'][b], p['bm'][b].reshape(1, F),
            p['Wra1'][b], p['bra1'][b].reshape(NRA, 1, F),
            p['Wra2'][b], p['bra2'][b].reshape(NRA, 1, F),
            p['Wro1'][b], p['bro1'][b].reshape(NRO, 1, F),
            p['Wro2'][b], p['bro2'][b].reshape(NRO, 1, F),
            woutp[b], boutp[b], wmixp[b], bmixp,
            p['Wi'][nb], p['bi'][nb].reshape(1, F),
            p['Wj'][nb], p['bj'][nb].reshape(1, F),
        )
        x, xi, xj, outacc, lo2, nh = _tc_block(
            x, xi, parts, outacc, lo2, nh, wts, gate=1.0 if b > 0 else 0.0)

    nz = p['scales'].shape[1]
    scp = jnp.zeros((F, F), jnp.float32).at[:nz, :NOUT].set(p['scales'].T)
    shp = jnp.zeros((F, F), jnp.float32).at[:nz, :NOUT].set(p['shifts'].T)
    res = _tc_final(outacc, Z.astype(jnp.int32).reshape(n, 1), scp, shp)
    return res[:, :NOUT], nh[0, 0]


# speedup vs baseline: 3.1948x; 1.4936x over previous
import functools

import numpy as np
import jax
import jax.numpy as jnp
from jax import lax
from jax.experimental import pallas as pl
from jax.experimental.pallas import tpu as pltpu
from jax.experimental.pallas import tpu_sc as plsc

F = 128
K = 64
NBLK = 5
NOUT = 2
NRI = 3
NRA = 2
NRO = 1
SR_CUT = 8.0

_NC = 2
_NS = 16
_NW = _NC * _NS
_L = 16

_HI = lax.Precision.HIGHEST
_LOG2 = np.float32(np.log(2.0))


def _act(x):
    return jnp.logaddexp(x, 0.0) - _LOG2


def _mm(a, w):
    return jnp.dot(a, w, preferred_element_type=jnp.float32, precision=_HI)


def _mmh(a, w):
    return jnp.dot(a, w, preferred_element_type=jnp.float32,
                   precision=lax.Precision.DEFAULT)


def _sc_d2(rx, ry, rz, ii, jj):
    n = rx.shape[0]
    e = ii.shape[0]
    ew = e // _NW
    mesh = plsc.VectorSubcoreMesh(core_axis_name="c", subcore_axis_name="s")

    @functools.partial(
        pl.kernel,
        out_type=jax.ShapeDtypeStruct((e,), jnp.float32),
        mesh=mesh,
        compiler_params=pltpu.CompilerParams(needs_layout_passes=False),
        scratch_types=[
            pltpu.VMEM((n,), jnp.float32),
            pltpu.VMEM((n,), jnp.float32),
            pltpu.VMEM((n,), jnp.float32),
            pltpu.VMEM((ew,), jnp.int32),
            pltpu.VMEM((ew,), jnp.int32),
            pltpu.VMEM((ew,), jnp.float32),
        ],
    )
    def k(rx_h, ry_h, rz_h, ii_h, jj_h, d2_h, rxv, ryv, rzv, iiv, jjv, d2v):
        wid = lax.axis_index("s") * _NC + lax.axis_index("c")
        base = wid * ew
        pltpu.sync_copy(rx_h, rxv)
        pltpu.sync_copy(ry_h, ryv)
        pltpu.sync_copy(rz_h, rzv)
        pltpu.sync_copy(ii_h.at[pl.ds(base, ew)], iiv)
        pltpu.sync_copy(jj_h.at[pl.ds(base, ew)], jjv)

        def body(t, carry):
            sl = pl.ds(t * _L, _L)
            ai = iiv[sl]
            aj = jjv[sl]
            dx = plsc.load_gather(rxv, [ai]) - plsc.load_gather(rxv, [aj])
            dy = plsc.load_gather(ryv, [ai]) - plsc.load_gather(ryv, [aj])
            dz = plsc.load_gather(rzv, [ai]) - plsc.load_gather(rzv, [aj])
            d2v[sl] = dx * dx + dy * dy + dz * dz
            return carry

        lax.fori_loop(0, ew // _L, body, 0)
        pltpu.sync_copy(d2v, d2_h.at[pl.ds(base, ew)])

    return k(rx, ry, rz, ii, jj)


def _sc_msg(gflat, xj, ii, jj, b):
    n = xj.shape[0]
    e = ii.shape[0]
    ew = e // _NW
    c = 80
    nchunk = ew // c
    rps = (n // _NS) // 8 * 8
    tail = n - rps * _NS
    mesh = plsc.VectorSubcoreMesh(core_axis_name="c", subcore_axis_name="s")

    @functools.partial(
        pl.kernel,
        out_type=jax.ShapeDtypeStruct((_NC, n, F), jnp.float32),
        mesh=mesh,
        compiler_params=pltpu.CompilerParams(needs_layout_passes=False),
        scratch_types=[
            pltpu.VMEM((c,), jnp.int32),
            pltpu.VMEM((c,), jnp.int32),
            pltpu.VMEM((c, F), jnp.float32),
            pltpu.VMEM((c, F), jnp.float32),
            pltpu.VMEM((c,), jnp.int32),
            pltpu.VMEM((c,), jnp.int32),
            pltpu.VMEM((c, F), jnp.float32),
            pltpu.VMEM((c, F), jnp.float32),
            pltpu.VMEM_SHARED((n, F), jnp.float32),
            pltpu.SemaphoreType.DMA,
            pltpu.SemaphoreType.DMA,
            pltpu.SemaphoreType.DMA,
            pltpu.SemaphoreType.DMA,
        ],
    )
    def k(g_h, xj_h, ii_h, jj_h, zero_h, out_h,
          iiva, jjva, rowsva, gva, iivb, jjvb, rowsvb, gvb, msh,
          semga, semca, semgb, semcb):
        cid = lax.axis_index("c")
        sid = lax.axis_index("s")
        wid = sid * _NC + cid

        pltpu.sync_copy(zero_h.at[pl.ds(sid * rps, rps)],
                        msh.at[pl.ds(sid * rps, rps)])
        if tail:
            @pl.when(sid == 0)
            def _():
                pltpu.sync_copy(zero_h.at[pl.ds(rps * _NS, tail)],
                                msh.at[pl.ds(rps * _NS, tail)])
        plsc.subcore_barrier()

        ebase = wid * ew

        def stage(kk, iiv, jjv, rowsv, gv, semg, semc):
            base = ebase + kk * c
            pltpu.sync_copy(ii_h.at[pl.ds(base, c)], iiv)
            pltpu.sync_copy(jj_h.at[pl.ds(base, c)], jjv)
            pltpu.async_copy(xj_h.at[jjv], rowsv, semg)
            pltpu.async_copy(g_h.at[pl.ds(b * e + base, c)], gv, semc)

        def process(kk, iiv, jjv, rowsv, gv, semg, semc):
            base = ebase + kk * c
            pltpu.make_async_copy(xj_h.at[jjv], rowsv, semg).wait()
            pltpu.make_async_copy(
                g_h.at[pl.ds(b * e + base, c)], gv, semc).wait()

            def mul(ee, cc):
                for f in range(F // _L):
                    sl = pl.ds(f * _L, _L)
                    rowsv[ee, sl] = rowsv[ee, sl] * gv[ee, sl]
                return cc

            lax.fori_loop(0, c, mul, 0)
            pltpu.sync_copy(rowsv, msh.at[iiv], add=True)

        bufa = (iiva, jjva, rowsva, gva, semga, semca)
        bufb = (iivb, jjvb, rowsvb, gvb, semgb, semcb)

        stage(0, *bufa)

        def pair(q, carry):
            k0 = 2 * q
            stage(k0 + 1, *bufb)
            process(k0, *bufa)
            stage(k0 + 2, *bufa)
            process(k0 + 1, *bufb)
            return carry

        lax.fori_loop(0, (nchunk - 1) // 2, pair, 0)
        process(nchunk - 1, *bufa)
        plsc.subcore_barrier()

        if tail:
            @pl.when(sid == 0)
            def _():
                pltpu.sync_copy(msh.at[pl.ds(rps * _NS, tail)],
                                out_h.at[cid, pl.ds(rps * _NS, tail)])
        pltpu.sync_copy(msh.at[pl.ds(sid * rps, rps)],
                        out_h.at[cid, pl.ds(sid * rps, rps)])

    zeros = jnp.zeros((n, F), jnp.float32)
    return k(gflat, xj, ii, jj, zeros)


_T = 2000


def _tc_edge_prep(d2sq):
    rows = d2sq.shape[0]

    def body(d2_ref, fc_ref, ed_ref):
        d = jnp.sqrt(d2_ref[...])
        r = d * np.float32(1.0 / SR_CUT)
        r2 = r * r
        r3 = r2 * r
        r4 = r2 * r2
        r5 = r4 * r
        fc_ref[...] = jnp.where(
            r < 1.0, 1.0 - 6.0 * r5 + 15.0 * r4 - 10.0 * r3, 0.0)
        ed_ref[...] = jnp.exp(-d)

    blk = pl.BlockSpec((rows, 128), lambda: (0, 0))
    return pl.pallas_call(
        body,
        in_specs=[blk],
        out_specs=[blk, blk],
        out_shape=[jax.ShapeDtypeStruct((rows, 128), jnp.float32)] * 2,
    )(d2sq)


def _tc_g(fcut, expd, gcat, centers):
    e = fcut.shape[0]
    nt = e // _T
    w = float(((2.0 / K) * (1.0 - np.exp(-SR_CUT))) ** (-2))

    def body(fc_ref, ed_ref, gc_ref, c_ref, o_ref, cat_ref):
        blk = pl.program_id(1)

        @pl.when(blk == 0)
        def _():
            diff = ed_ref[...] - c_ref[...]
            rbf = fc_ref[...] * jnp.exp(np.float32(-w) * (diff * diff))
            hi = rbf.astype(jnp.bfloat16)
            lo = (rbf - hi.astype(jnp.float32)).astype(jnp.bfloat16)
            cat_ref[...] = jnp.concatenate([hi, hi, lo, lo], axis=1)

        o_ref[...] = jnp.dot(cat_ref[...], gc_ref[0],
                             preferred_element_type=jnp.float32)

    return pl.pallas_call(
        body,
        grid=(nt, NBLK),
        in_specs=[
            pl.BlockSpec((_T, 1), lambda i, b: (i, 0)),
            pl.BlockSpec((_T, 1), lambda i, b: (i, 0)),
            pl.BlockSpec((1, 4 * K, F), lambda i, b: (b, 0, 0)),
            pl.BlockSpec((1, K), lambda i, b: (0, 0)),
        ],
        out_specs=pl.BlockSpec((_T, F), lambda i, b: (b * nt + i, 0)),
        out_shape=jax.ShapeDtypeStruct((NBLK * e, F), jnp.float32),
        scratch_shapes=[pltpu.VMEM((_T, 4 * K), jnp.bfloat16)],
    )(fcut, expd, gcat, centers)


def _tc_init(zf, mf, qa, qb, w0, b0, w1, b1, w2, b2, wi, bi, wj, bj):
    n = zf.shape[0]
    nt = n // _T

    def body(z_ref, m_ref, a_ref, q_ref, w0_ref, b0_ref, w1_ref, b1_ref,
             w2_ref, b2_ref, wi_ref, bi_ref, wj_ref, bj_ref,
             x_ref, xi_ref, xj_ref):
        pre = (z_ref[...] * w0_ref[0:1, :] + m_ref[...] * w0_ref[1:2, :]
               + a_ref[...] * w0_ref[2:3, :] + q_ref[...] * w0_ref[3:4, :]
               + b0_ref[...])
        x = _act(pre)
        x = _act(_mm(x, w1_ref[...]) + b1_ref[...])
        x = _mm(x, w2_ref[...]) + b2_ref[...]
        xt = _act(x)
        x_ref[...] = x
        xi_ref[...] = _act(_mm(xt, wi_ref[...]) + bi_ref[...])
        xj_ref[...] = _act(_mm(xt, wj_ref[...]) + bj_ref[...])

    col = pl.BlockSpec((_T, 1), lambda i: (i, 0))
    row = lambda r: pl.BlockSpec((r, F), lambda i: (0, 0))
    tile = pl.BlockSpec((_T, F), lambda i: (i, 0))
    return pl.pallas_call(
        body,
        grid=(nt,),
        in_specs=[col, col, col, col,
                  row(4), row(1), row(F), row(1), row(F), row(1),
                  row(F), row(1), row(F), row(1)],
        out_specs=[tile, tile, tile],
        out_shape=[jax.ShapeDtypeStruct((n, F), jnp.float32)] * 3,
    )(zf, mf, qa, qb, w0, b0, w1, b1, w2, b2, wi, bi, wj, bj)


def _tc_block(x, xi, parts, outacc, lo2, nh, wts, gate):
    n = x.shape[0]
    nt = n // _T
    inv = float(gate) / float(n * NOUT)

    def body(x_ref, xi_ref, p_ref, oa_ref, lo_ref, nh_ref,
             wri1, bri1, wri2, bri2, wm, bm, wra1, bra1, wra2, bra2,
             wro1, bro1, wro2, bro2, wout, bout, wmix, bmix,
             win, bin_, wjn, bjn,
             xo_ref, xio_ref, xjo_ref, oao_ref, loo_ref, nho_ref):
        m = xi_ref[...] + p_ref[0] + p_ref[1]
        for t in range(NRI):
            m = m + _mm(_act(_mm(_act(m), wri1[t]) + bri1[t]), wri2[t]) + bri2[t]
        x = x_ref[...] + _mm(_act(m), wm[...]) + bm[...]
        for t in range(NRA):
            x = x + _mm(_act(_mm(_act(x), wra1[t]) + bra1[t]), wra2[t]) + bra2[t]
        o = x
        for t in range(NRO):
            o = o + _mm(_act(_mm(_act(o), wro1[t]) + bro1[t]), wro2[t]) + bro2[t]
        out = _mm(_act(o), wout[...]) + bout[...]
        oao_ref[...] = oa_ref[...] + out * wmix[...] + bmix[...]
        out2 = out * out
        frac = out2 / (out2 + lo_ref[...] + np.float32(1e-7))
        loo_ref[...] = out2
        part = jnp.sum(frac) * np.float32(inv)

        @pl.when(pl.program_id(0) == 0)
        def _():
            nho_ref[...] = nh_ref[...] + part

        @pl.when(pl.program_id(0) != 0)
        def _():
            nho_ref[...] = nho_ref[...] + part

        xo_ref[...] = x
        xt = _act(x)
        xio_ref[...] = _act(_mm(xt, win[...]) + bin_[...])
        xjo_ref[...] = _act(_mm(xt, wjn[...]) + bjn[...])

    tile = pl.BlockSpec((_T, F), lambda i: (i, 0))
    scal = pl.BlockSpec((1, 1), lambda i: (0, 0))

    def wspec(a):
        nd = a.ndim
        return pl.BlockSpec(a.shape, lambda i, _nd=nd: (0,) * _nd)

    (wri1, bri1, wri2, bri2, wm, bm, wra1, bra1, wra2, bra2,
     wro1, bro1, wro2, bro2, wout, bout, wmix, bmix,
     win, bin_, wjn, bjn) = wts
    pspec = pl.BlockSpec((_NC, _T, F), lambda i: (0, i, 0))
    return pl.pallas_call(
        body,
        grid=(nt,),
        in_specs=[tile, tile, pspec, tile, tile, scal] + [wspec(a) for a in wts],
        out_specs=[tile, tile, tile, tile, tile, scal],
        out_shape=[jax.ShapeDtypeStruct((n, F), jnp.float32)] * 5
        + [jax.ShapeDtypeStruct((1, 1), jnp.float32)],
    )(x, xi, parts, outacc, lo2, nh, wri1, bri1, wri2, bri2, wm, bm,
      wra1, bra1, wra2, bra2, wro1, bro1, wro2, bro2, wout, bout,
      wmix, bmix, win, bin_, wjn, bjn)


def _tc_final(outp, zcol, scp, shp):
    n = outp.shape[0]
    nt = n // _T

    def body(o_ref, z_ref, sc_ref, sh_ref, r_ref):
        ids = lax.broadcasted_iota(jnp.int32, (_T, F), 1)
        oh = (ids == z_ref[...]).astype(jnp.float32)
        sc = _mm(oh, sc_ref[...])
        sh = _mm(oh, sh_ref[...])
        r_ref[...] = o_ref[...] * sc + sh

    tile = pl.BlockSpec((_T, F), lambda i: (i, 0))
    col = pl.BlockSpec((_T, 1), lambda i: (i, 0))
    full = pl.BlockSpec((F, F), lambda i: (0, 0))
    return pl.pallas_call(
        body,
        grid=(nt,),
        in_specs=[tile, col, full, full],
        out_specs=tile,
        out_shape=jax.ShapeDtypeStruct((n, F), jnp.float32),
    )(outp, zcol, scp, shp)


def kernel(Z, R, idx_i, idx_j, M, QaAlpha, QaBeta, batch_seg, params):
    n = Z.shape[0]
    e = idx_i.shape[0]
    p = params

    ii = idx_i.astype(jnp.int32)
    jj = idx_j.astype(jnp.int32)
    rx = R[:, 0]
    ry = R[:, 1]
    rz = R[:, 2]

    d2 = _sc_d2(rx, ry, rz, ii, jj)

    centers = jnp.linspace(
        np.float32(np.exp(-SR_CUT)), 1.0, K).astype(jnp.float32).reshape(1, K)
    fcut, expd = _tc_edge_prep(d2.reshape(e // 128, 128))
    ghi = p['G'].astype(jnp.bfloat16)
    glo = (p['G'] - ghi.astype(jnp.float32)).astype(jnp.bfloat16)
    gcat = jnp.concatenate([ghi, glo, ghi, glo], axis=1)
    gflat = _tc_g(fcut.reshape(e, 1), expd.reshape(e, 1), gcat, centers)

    zf = Z.astype(jnp.float32).reshape(n, 1)
    x, xi, xj = _tc_init(
        zf, M.reshape(n, 1), QaAlpha.reshape(n, 1), QaBeta.reshape(n, 1),
        p['Wem0'], p['bem0'].reshape(1, F),
        p['Wem1'], p['bem1'].reshape(1, F),
        p['Wem2'], p['bem2'].reshape(1, F),
        p['Wi'][0], p['bi'][0].reshape(1, F),
        p['Wj'][0], p['bj'][0].reshape(1, F))

    woutp = jnp.zeros((NBLK, F, F), jnp.float32).at[:, :, :NOUT].set(p['Wout'])
    boutp = jnp.zeros((NBLK, 1, F), jnp.float32).at[:, 0, :NOUT].set(p['bout'])
    wmixp = jnp.zeros((NBLK, 1, F), jnp.float32).at[:, 0, :NOUT].set(p['Wmix'])
    bmixp = jnp.zeros((1, F), jnp.float32).at[0, :NOUT].set(p['bmix'])

    outacc = jnp.zeros((n, F), jnp.float32)
    lo2 = jnp.zeros((n, F), jnp.float32)
    nh = jnp.zeros((1, 1), jnp.float32)

    for b in range(NBLK):
        parts = _sc_msg(gflat, xj, ii, jj, b)
        nb = (b + 1) % NBLK
        wts = (
            p['Wri1'][b], p['bri1'][b].reshape(NRI, 1, F),
            p['Wri2'][b], p['bri2'][b].reshape(NRI, 1, F),
            p['Wm'][b], p['bm'][b].reshape(1, F),
            p['Wra1'][b], p['bra1'][b].reshape(NRA, 1, F),
            p['Wra2'][b], p['bra2'][b].reshape(NRA, 1, F),
            p['Wro1'][b], p['bro1'][b].reshape(NRO, 1, F),
            p['Wro2'][b], p['bro2'][b].reshape(NRO, 1, F),
            woutp[b], boutp[b], wmixp[b], bmixp,
            p['Wi'][nb], p['bi'][nb].reshape(1, F),
            p['Wj'][nb], p['bj'][nb].reshape(1, F),
        )
        x, xi, xj, outacc, lo2, nh = _tc_block(
            x, xi, parts, outacc, lo2, nh, wts, gate=1.0 if b > 0 else 0.0)

    nz = p['scales'].shape[1]
    scp = jnp.zeros((F, F), jnp.float32).at[:nz, :NOUT].set(p['scales'].T)
    shp = jnp.zeros((F, F), jnp.float32).at[:nz, :NOUT].set(p['shifts'].T)
    res = _tc_final(outacc, Z.astype(jnp.int32).reshape(n, 1), scp, shp)
    return res[:, :NOUT], nh[0, 0]


# speedup vs baseline: 3.4186x; 1.0700x over previous
import functools

import numpy as np
import jax
import jax.numpy as jnp
from jax import lax
from jax.experimental import pallas as pl
from jax.experimental.pallas import tpu as pltpu
from jax.experimental.pallas import tpu_sc as plsc

F = 128
K = 64
NBLK = 5
NOUT = 2
NRI = 3
NRA = 2
NRO = 1
SR_CUT = 8.0

_NC = 2
_NS = 16
_NW = _NC * _NS
_L = 16

_HI = lax.Precision.HIGHEST
_LOG2 = np.float32(np.log(2.0))


def _act(x):
    return jnp.logaddexp(x, 0.0) - _LOG2


def _mm(a, w):
    return jnp.dot(a, w, preferred_element_type=jnp.float32, precision=_HI)


def _mmh(a, w):
    return jnp.dot(a, w, preferred_element_type=jnp.float32,
                   precision=lax.Precision.DEFAULT)


def _sc_d2(rx, ry, rz, ii, jj):
    n = rx.shape[0]
    e = ii.shape[0]
    ew = e // _NW
    mesh = plsc.VectorSubcoreMesh(core_axis_name="c", subcore_axis_name="s")

    @functools.partial(
        pl.kernel,
        out_type=jax.ShapeDtypeStruct((e,), jnp.float32),
        mesh=mesh,
        compiler_params=pltpu.CompilerParams(needs_layout_passes=False),
        scratch_types=[
            pltpu.VMEM((n,), jnp.float32),
            pltpu.VMEM((n,), jnp.float32),
            pltpu.VMEM((n,), jnp.float32),
            pltpu.VMEM((ew,), jnp.int32),
            pltpu.VMEM((ew,), jnp.int32),
            pltpu.VMEM((ew,), jnp.float32),
        ],
    )
    def k(rx_h, ry_h, rz_h, ii_h, jj_h, d2_h, rxv, ryv, rzv, iiv, jjv, d2v):
        wid = lax.axis_index("s") * _NC + lax.axis_index("c")
        base = wid * ew
        pltpu.sync_copy(rx_h, rxv)
        pltpu.sync_copy(ry_h, ryv)
        pltpu.sync_copy(rz_h, rzv)
        pltpu.sync_copy(ii_h.at[pl.ds(base, ew)], iiv)
        pltpu.sync_copy(jj_h.at[pl.ds(base, ew)], jjv)

        def body(t, carry):
            sl = pl.ds(t * _L, _L)
            ai = iiv[sl]
            aj = jjv[sl]
            dx = plsc.load_gather(rxv, [ai]) - plsc.load_gather(rxv, [aj])
            dy = plsc.load_gather(ryv, [ai]) - plsc.load_gather(ryv, [aj])
            dz = plsc.load_gather(rzv, [ai]) - plsc.load_gather(rzv, [aj])
            d2v[sl] = dx * dx + dy * dy + dz * dz
            return carry

        lax.fori_loop(0, ew // _L, body, 0)
        pltpu.sync_copy(d2v, d2_h.at[pl.ds(base, ew)])

    return k(rx, ry, rz, ii, jj)


def _sc_msg(gflat, xj, ii, jj, b):
    n = xj.shape[0]
    e = ii.shape[0]
    ew = e // _NW
    c = 80
    nchunk = ew // c
    rps = (n // _NS) // 8 * 8
    tail = n - rps * _NS
    mesh = plsc.VectorSubcoreMesh(core_axis_name="c", subcore_axis_name="s")

    @functools.partial(
        pl.kernel,
        out_type=jax.ShapeDtypeStruct((_NC, n, F), jnp.float32),
        mesh=mesh,
        compiler_params=pltpu.CompilerParams(needs_layout_passes=False),
        scratch_types=[
            pltpu.VMEM((ew,), jnp.int32),
            pltpu.VMEM((c,), jnp.int32),
            pltpu.VMEM((c,), jnp.int32),
            pltpu.VMEM((c,), jnp.int32),
            pltpu.VMEM((c,), jnp.int32),
            pltpu.VMEM((c, F), jnp.float32),
            pltpu.VMEM((c, F), jnp.float32),
            pltpu.VMEM((c, F), jnp.float32),
            pltpu.VMEM_SHARED((n, F), jnp.float32),
            pltpu.SemaphoreType.DMA,
            pltpu.SemaphoreType.DMA,
            pltpu.SemaphoreType.DMA,
            pltpu.SemaphoreType.DMA,
        ],
    )
    def k(g_h, xj_h, ii_h, jj_h, zero_h, out_h,
          jjv, jjsa, jjsb, iisa, iisb, rowsva, rowsvb, gv, msh,
          semga, semgb, semia, semib):
        cid = lax.axis_index("c")
        sid = lax.axis_index("s")
        wid = sid * _NC + cid
        ebase = wid * ew

        pltpu.sync_copy(jj_h.at[pl.ds(ebase, ew)], jjv)

        pltpu.sync_copy(zero_h.at[pl.ds(sid * rps, rps)],
                        msh.at[pl.ds(sid * rps, rps)])
        if tail:
            @pl.when(sid == 0)
            def _():
                pltpu.sync_copy(zero_h.at[pl.ds(rps * _NS, tail)],
                                msh.at[pl.ds(rps * _NS, tail)])
        plsc.subcore_barrier()

        def fire(kk, rowsv, semg, jjs, iisx, semi):
            for f in range(c // _L):
                jjs[pl.ds(f * _L, _L)] = jjv[pl.ds(kk * c + f * _L, _L)]
            pltpu.async_copy(xj_h.at[jjs], rowsv, semg)
            pltpu.async_copy(ii_h.at[pl.ds(ebase + kk * c, c)], iisx, semi)

        def process(kk, rowsv, semg, jjs, iisx, semi):
            pltpu.sync_copy(g_h.at[pl.ds(b * e + ebase + kk * c, c)], gv)
            pltpu.make_async_copy(
                ii_h.at[pl.ds(ebase + kk * c, c)], iisx, semi).wait()
            pltpu.make_async_copy(xj_h.at[jjs], rowsv, semg).wait()

            def mul(ee, cc):
                for f in range(F // _L):
                    sl = pl.ds(f * _L, _L)
                    rowsv[ee, sl] = rowsv[ee, sl] * gv[ee, sl]
                return cc

            lax.fori_loop(0, c, mul, 0)
            pltpu.sync_copy(rowsv, msh.at[iisx], add=True)

        bufa = (rowsva, semga, jjsa, iisa, semia)
        bufb = (rowsvb, semgb, jjsb, iisb, semib)

        fire(0, *bufa)

        def pair(q, carry):
            k0 = 2 * q
            fire(k0 + 1, *bufb)
            process(k0, *bufa)
            fire(k0 + 2, *bufa)
            process(k0 + 1, *bufb)
            return carry

        lax.fori_loop(0, (nchunk - 1) // 2, pair, 0)
        process(nchunk - 1, *bufa)
        plsc.subcore_barrier()

        if tail:
            @pl.when(sid == 0)
            def _():
                pltpu.sync_copy(msh.at[pl.ds(rps * _NS, tail)],
                                out_h.at[cid, pl.ds(rps * _NS, tail)])
        pltpu.sync_copy(msh.at[pl.ds(sid * rps, rps)],
                        out_h.at[cid, pl.ds(sid * rps, rps)])

    zeros = jnp.zeros((n, F), jnp.float32)
    return k(gflat, xj, ii, jj, zeros)


_T = 2000


_TE = 2560


def _tc_edge_prep(d2sq):
    rows = d2sq.shape[0]

    def body(d2_ref, fc_ref, ed_ref):
        d = jnp.sqrt(d2_ref[...])
        r = d * np.float32(1.0 / SR_CUT)
        r2 = r * r
        r3 = r2 * r
        r4 = r2 * r2
        r5 = r4 * r
        fc_ref[...] = jnp.where(
            r < 1.0, 1.0 - 6.0 * r5 + 15.0 * r4 - 10.0 * r3, 0.0)
        ed_ref[...] = jnp.exp(-d)

    blk = pl.BlockSpec((rows, _TE), lambda: (0, 0))
    return pl.pallas_call(
        body,
        in_specs=[blk],
        out_specs=[blk, blk],
        out_shape=[jax.ShapeDtypeStruct((rows, _TE), jnp.float32)] * 2,
    )(d2sq)


def _tc_g(fcut, expd, gcat, centers):
    nt = fcut.shape[0]
    e = nt * _TE
    w = float(((2.0 / K) * (1.0 - np.exp(-SR_CUT))) ** (-2))

    def body(fc_ref, ed_ref, gc_ref, c_ref, o_ref, cat_ref):
        blk = pl.program_id(1)

        @pl.when(blk == 0)
        def _():
            diff = ed_ref[0] - c_ref[...]
            rbf = fc_ref[0] * jnp.exp(np.float32(-w) * (diff * diff))
            hi = rbf.astype(jnp.bfloat16)
            lo = (rbf - hi.astype(jnp.float32)).astype(jnp.bfloat16)
            cat_ref[...] = jnp.concatenate([hi, hi, lo, lo], axis=0)

        o_ref[...] = lax.dot_general(
            cat_ref[...], gc_ref[0],
            dimension_numbers=(((0,), (0,)), ((), ())),
            preferred_element_type=jnp.float32)

    return pl.pallas_call(
        body,
        grid=(nt, NBLK),
        in_specs=[
            pl.BlockSpec((1, 1, _TE), lambda i, b: (i, 0, 0)),
            pl.BlockSpec((1, 1, _TE), lambda i, b: (i, 0, 0)),
            pl.BlockSpec((1, 4 * K, F), lambda i, b: (b, 0, 0)),
            pl.BlockSpec((K, 1), lambda i, b: (0, 0)),
        ],
        out_specs=pl.BlockSpec((_TE, F), lambda i, b: (b * nt + i, 0)),
        out_shape=jax.ShapeDtypeStruct((NBLK * e, F), jnp.float32),
        scratch_shapes=[pltpu.VMEM((4 * K, _TE), jnp.bfloat16)],
    )(fcut, expd, gcat, centers)


def _tc_init(zf, mf, qa, qb, w0, b0, w1, b1, w2, b2, wi, bi, wj, bj):
    n = zf.shape[0]
    nt = n // _T

    def body(z_ref, m_ref, a_ref, q_ref, w0_ref, b0_ref, w1_ref, b1_ref,
             w2_ref, b2_ref, wi_ref, bi_ref, wj_ref, bj_ref,
             x_ref, xi_ref, xj_ref):
        pre = (z_ref[...] * w0_ref[0:1, :] + m_ref[...] * w0_ref[1:2, :]
               + a_ref[...] * w0_ref[2:3, :] + q_ref[...] * w0_ref[3:4, :]
               + b0_ref[...])
        x = _act(pre)
        x = _act(_mm(x, w1_ref[...]) + b1_ref[...])
        x = _mm(x, w2_ref[...]) + b2_ref[...]
        xt = _act(x)
        x_ref[...] = x
        xi_ref[...] = _act(_mm(xt, wi_ref[...]) + bi_ref[...])
        xj_ref[...] = _act(_mm(xt, wj_ref[...]) + bj_ref[...])

    col = pl.BlockSpec((_T, 1), lambda i: (i, 0))
    row = lambda r: pl.BlockSpec((r, F), lambda i: (0, 0))
    tile = pl.BlockSpec((_T, F), lambda i: (i, 0))
    return pl.pallas_call(
        body,
        grid=(nt,),
        in_specs=[col, col, col, col,
                  row(4), row(1), row(F), row(1), row(F), row(1),
                  row(F), row(1), row(F), row(1)],
        out_specs=[tile, tile, tile],
        out_shape=[jax.ShapeDtypeStruct((n, F), jnp.float32)] * 3,
    )(zf, mf, qa, qb, w0, b0, w1, b1, w2, b2, wi, bi, wj, bj)


def _tc_block(x, xi, parts, outacc, lo2, nh, wts, gate):
    n = x.shape[0]
    nt = n // _T
    inv = float(gate) / float(n * NOUT)

    def body(x_ref, xi_ref, p_ref, oa_ref, lo_ref, nh_ref,
             wri1, bri1, wri2, bri2, wm, bm, wra1, bra1, wra2, bra2,
             wro1, bro1, wro2, bro2, wout, bout, wmix, bmix,
             win, bin_, wjn, bjn,
             xo_ref, xio_ref, xjo_ref, oao_ref, loo_ref, nho_ref):
        m = xi_ref[...] + p_ref[0] + p_ref[1]
        for t in range(NRI):
            m = m + _mm(_act(_mm(_act(m), wri1[t]) + bri1[t]), wri2[t]) + bri2[t]
        x = x_ref[...] + _mm(_act(m), wm[...]) + bm[...]
        for t in range(NRA):
            x = x + _mm(_act(_mm(_act(x), wra1[t]) + bra1[t]), wra2[t]) + bra2[t]
        o = x
        for t in range(NRO):
            o = o + _mm(_act(_mm(_act(o), wro1[t]) + bro1[t]), wro2[t]) + bro2[t]
        out = _mm(_act(o), wout[...]) + bout[...]
        oao_ref[...] = oa_ref[...] + out * wmix[...] + bmix[...]
        out2 = out * out
        frac = out2 / (out2 + lo_ref[...] + np.float32(1e-7))
        loo_ref[...] = out2
        part = jnp.sum(frac) * np.float32(inv)

        @pl.when(pl.program_id(0) == 0)
        def _():
            nho_ref[...] = nh_ref[...] + part

        @pl.when(pl.program_id(0) != 0)
        def _():
            nho_ref[...] = nho_ref[...] + part

        xo_ref[...] = x
        xt = _act(x)
        xio_ref[...] = _act(_mm(xt, win[...]) + bin_[...])
        xjo_ref[...] = _act(_mm(xt, wjn[...]) + bjn[...])

    tile = pl.BlockSpec((_T, F), lambda i: (i, 0))
    scal = pl.BlockSpec((1, 1), lambda i: (0, 0))

    def wspec(a):
        nd = a.ndim
        return pl.BlockSpec(a.shape, lambda i, _nd=nd: (0,) * _nd)

    (wri1, bri1, wri2, bri2, wm, bm, wra1, bra1, wra2, bra2,
     wro1, bro1, wro2, bro2, wout, bout, wmix, bmix,
     win, bin_, wjn, bjn) = wts
    pspec = pl.BlockSpec((_NC, _T, F), lambda i: (0, i, 0))
    return pl.pallas_call(
        body,
        grid=(nt,),
        in_specs=[tile, tile, pspec, tile, tile, scal] + [wspec(a) for a in wts],
        out_specs=[tile, tile, tile, tile, tile, scal],
        out_shape=[jax.ShapeDtypeStruct((n, F), jnp.float32)] * 5
        + [jax.ShapeDtypeStruct((1, 1), jnp.float32)],
    )(x, xi, parts, outacc, lo2, nh, wri1, bri1, wri2, bri2, wm, bm,
      wra1, bra1, wra2, bra2, wro1, bro1, wro2, bro2, wout, bout,
      wmix, bmix, win, bin_, wjn, bjn)


def _tc_final(outp, zcol, scp, shp):
    n = outp.shape[0]
    nt = n // _T

    def body(o_ref, z_ref, sc_ref, sh_ref, r_ref):
        ids = lax.broadcasted_iota(jnp.int32, (_T, F), 1)
        oh = (ids == z_ref[...]).astype(jnp.float32)
        sc = _mm(oh, sc_ref[...])
        sh = _mm(oh, sh_ref[...])
        r_ref[...] = o_ref[...] * sc + sh

    tile = pl.BlockSpec((_T, F), lambda i: (i, 0))
    col = pl.BlockSpec((_T, 1), lambda i: (i, 0))
    full = pl.BlockSpec((F, F), lambda i: (0, 0))
    return pl.pallas_call(
        body,
        grid=(nt,),
        in_specs=[tile, col, full, full],
        out_specs=tile,
        out_shape=jax.ShapeDtypeStruct((n, F), jnp.float32),
    )(outp, zcol, scp, shp)


def kernel(Z, R, idx_i, idx_j, M, QaAlpha, QaBeta, batch_seg, params):
    n = Z.shape[0]
    e = idx_i.shape[0]
    p = params

    ii = idx_i.astype(jnp.int32)
    jj = idx_j.astype(jnp.int32)
    rx = R[:, 0]
    ry = R[:, 1]
    rz = R[:, 2]

    d2 = _sc_d2(rx, ry, rz, ii, jj)

    centers = jnp.linspace(
        np.float32(np.exp(-SR_CUT)), 1.0, K).astype(jnp.float32).reshape(1, K)
    fcut, expd = _tc_edge_prep(d2.reshape(e // _TE, _TE))
    ghi = p['G'].astype(jnp.bfloat16)
    glo = (p['G'] - ghi.astype(jnp.float32)).astype(jnp.bfloat16)
    gcat = jnp.concatenate([ghi, glo, ghi, glo], axis=1)
    gflat = _tc_g(fcut.reshape(e // _TE, 1, _TE),
                  expd.reshape(e // _TE, 1, _TE), gcat, centers.reshape(K, 1))

    zf = Z.astype(jnp.float32).reshape(n, 1)
    x, xi, xj = _tc_init(
        zf, M.reshape(n, 1), QaAlpha.reshape(n, 1), QaBeta.reshape(n, 1),
        p['Wem0'], p['bem0'].reshape(1, F),
        p['Wem1'], p['bem1'].reshape(1, F),
        p['Wem2'], p['bem2'].reshape(1, F),
        p['Wi'][0], p['bi'][0].reshape(1, F),
        p['Wj'][0], p['bj'][0].reshape(1, F))

    woutp = jnp.zeros((NBLK, F, F), jnp.float32).at[:, :, :NOUT].set(p['Wout'])
    boutp = jnp.zeros((NBLK, 1, F), jnp.float32).at[:, 0, :NOUT].set(p['bout'])
    wmixp = jnp.zeros((NBLK, 1, F), jnp.float32).at[:, 0, :NOUT].set(p['Wmix'])
    bmixp = jnp.zeros((1, F), jnp.float32).at[0, :NOUT].set(p['bmix'])

    outacc = jnp.zeros((n, F), jnp.float32)
    lo2 = jnp.zeros((n, F), jnp.float32)
    nh = jnp.zeros((1, 1), jnp.float32)

    for b in range(NBLK):
        parts = _sc_msg(gflat, xj, ii, jj, b)
        nb = (b + 1) % NBLK
        wts = (
            p['Wri1'][b], p['bri1'][b].reshape(NRI, 1, F),
            p['Wri2'][b], p['bri2'][b].reshape(NRI, 1, F),
            p['Wm'][b], p['bm'][b].reshape(1, F),
            p['Wra1'][b], p['bra1'][b].reshape(NRA, 1, F),
            p['Wra2'][b], p['bra2'][b].reshape(NRA, 1, F),
            p['Wro1'][b], p['bro1'][b].reshape(NRO, 1, F),
            p['Wro2'][b], p['bro2'][b].reshape(NRO, 1, F),
            woutp[b], boutp[b], wmixp[b], bmixp,
            p['Wi'][nb], p['bi'][nb].reshape(1, F),
            p['Wj'][nb], p['bj'][nb].reshape(1, F),
        )
        x, xi, xj, outacc, lo2, nh = _tc_block(
            x, xi, parts, outacc, lo2, nh, wts, gate=1.0 if b > 0 else 0.0)

    nz = p['scales'].shape[1]
    scp = jnp.zeros((F, F), jnp.float32).at[:nz, :NOUT].set(p['scales'].T)
    shp = jnp.zeros((F, F), jnp.float32).at[:nz, :NOUT].set(p['shifts'].T)
    res = _tc_final(outacc, Z.astype(jnp.int32).reshape(n, 1), scp, shp)
    return res[:, :NOUT], nh[0, 0]


# speedup vs baseline: 3.5985x; 1.0526x over previous
import functools

import numpy as np
import jax
import jax.numpy as jnp
from jax import lax
from jax.experimental import pallas as pl
from jax.experimental.pallas import tpu as pltpu
from jax.experimental.pallas import tpu_sc as plsc

F = 128
K = 64
NBLK = 5
NOUT = 2
NRI = 3
NRA = 2
NRO = 1
SR_CUT = 8.0

_NC = 2
_NS = 16
_NW = _NC * _NS
_L = 16

_HI = lax.Precision.HIGHEST
_LOG2 = np.float32(np.log(2.0))


def _act(x):
    return jnp.logaddexp(x, 0.0) - _LOG2


def _mm(a, w):
    return jnp.dot(a, w, preferred_element_type=jnp.float32, precision=_HI)


def _mmh(a, w):
    return jnp.dot(a, w, preferred_element_type=jnp.float32,
                   precision=lax.Precision.DEFAULT)


def _sc_d2(rx, ry, rz, ii, jj):
    n = rx.shape[0]
    e = ii.shape[0]
    ew = e // _NW
    mesh = plsc.VectorSubcoreMesh(core_axis_name="c", subcore_axis_name="s")

    @functools.partial(
        pl.kernel,
        out_type=jax.ShapeDtypeStruct((e,), jnp.float32),
        mesh=mesh,
        compiler_params=pltpu.CompilerParams(needs_layout_passes=False),
        scratch_types=[
            pltpu.VMEM((n,), jnp.float32),
            pltpu.VMEM((n,), jnp.float32),
            pltpu.VMEM((n,), jnp.float32),
            pltpu.VMEM((ew,), jnp.int32),
            pltpu.VMEM((ew,), jnp.int32),
            pltpu.VMEM((ew,), jnp.float32),
        ],
    )
    def k(rx_h, ry_h, rz_h, ii_h, jj_h, d2_h, rxv, ryv, rzv, iiv, jjv, d2v):
        wid = lax.axis_index("s") * _NC + lax.axis_index("c")
        base = wid * ew
        pltpu.sync_copy(rx_h, rxv)
        pltpu.sync_copy(ry_h, ryv)
        pltpu.sync_copy(rz_h, rzv)
        pltpu.sync_copy(ii_h.at[pl.ds(base, ew)], iiv)
        pltpu.sync_copy(jj_h.at[pl.ds(base, ew)], jjv)

        def body(t, carry):
            sl = pl.ds(t * _L, _L)
            ai = iiv[sl]
            aj = jjv[sl]
            dx = plsc.load_gather(rxv, [ai]) - plsc.load_gather(rxv, [aj])
            dy = plsc.load_gather(ryv, [ai]) - plsc.load_gather(ryv, [aj])
            dz = plsc.load_gather(rzv, [ai]) - plsc.load_gather(rzv, [aj])
            d2v[sl] = dx * dx + dy * dy + dz * dz
            return carry

        lax.fori_loop(0, ew // _L, body, 0)
        pltpu.sync_copy(d2v, d2_h.at[pl.ds(base, ew)])

    return k(rx, ry, rz, ii, jj)


def _sc_msg(gflat, xj, ii, jj, b):
    n = xj.shape[0]
    e = ii.shape[0]
    ew = e // _NW
    c = 80
    nchunk = ew // c
    rps = (n // _NS) // 8 * 8
    tail = n - rps * _NS
    mesh = plsc.VectorSubcoreMesh(core_axis_name="c", subcore_axis_name="s")

    @functools.partial(
        pl.kernel,
        out_type=jax.ShapeDtypeStruct((_NC, n, F), jnp.float32),
        mesh=mesh,
        compiler_params=pltpu.CompilerParams(needs_layout_passes=False),
        scratch_types=[
            pltpu.VMEM((ew,), jnp.int32),
            pltpu.VMEM((c,), jnp.int32),
            pltpu.VMEM((c,), jnp.int32),
            pltpu.VMEM((c,), jnp.int32),
            pltpu.VMEM((c,), jnp.int32),
            pltpu.VMEM((c, F), jnp.float32),
            pltpu.VMEM((c, F), jnp.float32),
            pltpu.VMEM((c, F), jnp.float32),
            pltpu.VMEM_SHARED((n, F), jnp.float32),
            pltpu.SemaphoreType.DMA,
            pltpu.SemaphoreType.DMA,
            pltpu.SemaphoreType.DMA,
            pltpu.SemaphoreType.DMA,
        ],
    )
    def k(g_h, xj_h, ii_h, jj_h, zero_h, out_h,
          jjv, jjsa, jjsb, iisa, iisb, rowsva, rowsvb, gv, msh,
          semga, semgb, semia, semib):
        cid = lax.axis_index("c")
        sid = lax.axis_index("s")
        wid = sid * _NC + cid
        ebase = wid * ew

        pltpu.sync_copy(jj_h.at[pl.ds(ebase, ew)], jjv)

        pltpu.sync_copy(zero_h.at[pl.ds(sid * rps, rps)],
                        msh.at[pl.ds(sid * rps, rps)])
        if tail:
            @pl.when(sid == 0)
            def _():
                pltpu.sync_copy(zero_h.at[pl.ds(rps * _NS, tail)],
                                msh.at[pl.ds(rps * _NS, tail)])
        plsc.subcore_barrier()

        def fire(kk, rowsv, semg, jjs, iisx, semi):
            for f in range(c // _L):
                jjs[pl.ds(f * _L, _L)] = jjv[pl.ds(kk * c + f * _L, _L)]
            pltpu.async_copy(xj_h.at[jjs], rowsv, semg)
            pltpu.async_copy(ii_h.at[pl.ds(ebase + kk * c, c)], iisx, semi)

        def process(kk, rowsv, semg, jjs, iisx, semi):
            pltpu.sync_copy(g_h.at[pl.ds(b * e + ebase + kk * c, c)], gv)
            pltpu.make_async_copy(
                ii_h.at[pl.ds(ebase + kk * c, c)], iisx, semi).wait()
            pltpu.make_async_copy(xj_h.at[jjs], rowsv, semg).wait()

            def mul(ee, cc):
                for f in range(F // _L):
                    sl = pl.ds(f * _L, _L)
                    rowsv[ee, sl] = rowsv[ee, sl] * gv[ee, sl]
                return cc

            lax.fori_loop(0, c, mul, 0)
            pltpu.sync_copy(rowsv, msh.at[iisx], add=True)

        bufa = (rowsva, semga, jjsa, iisa, semia)
        bufb = (rowsvb, semgb, jjsb, iisb, semib)

        fire(0, *bufa)

        def pair(q, carry):
            k0 = 2 * q
            fire(k0 + 1, *bufb)
            process(k0, *bufa)
            fire(k0 + 2, *bufa)
            process(k0 + 1, *bufb)
            return carry

        lax.fori_loop(0, (nchunk - 1) // 2, pair, 0)
        process(nchunk - 1, *bufa)
        plsc.subcore_barrier()

        if tail:
            @pl.when(sid == 0)
            def _():
                pltpu.sync_copy(msh.at[pl.ds(rps * _NS, tail)],
                                out_h.at[cid, pl.ds(rps * _NS, tail)])
        pltpu.sync_copy(msh.at[pl.ds(sid * rps, rps)],
                        out_h.at[cid, pl.ds(sid * rps, rps)])

    zeros = jnp.zeros((n, F), jnp.float32)
    return k(gflat, xj, ii, jj, zeros)


_T = 2000


_TE = 2560


def _tc_edge_prep(d2sq):
    rows = d2sq.shape[0]

    def body(d2_ref, fc_ref, ed_ref):
        d = jnp.sqrt(d2_ref[...])
        r = d * np.float32(1.0 / SR_CUT)
        r2 = r * r
        r3 = r2 * r
        r4 = r2 * r2
        r5 = r4 * r
        fc_ref[...] = jnp.where(
            r < 1.0, 1.0 - 6.0 * r5 + 15.0 * r4 - 10.0 * r3, 0.0)
        ed_ref[...] = jnp.exp(-d)

    blk = pl.BlockSpec((rows, _TE), lambda: (0, 0))
    return pl.pallas_call(
        body,
        in_specs=[blk],
        out_specs=[blk, blk],
        out_shape=[jax.ShapeDtypeStruct((rows, _TE), jnp.float32)] * 2,
    )(d2sq)


def _tc_g(fcut, expd, gcat, centers):
    nt = fcut.shape[0]
    e = nt * _TE
    w = float(((2.0 / K) * (1.0 - np.exp(-SR_CUT))) ** (-2))

    def body(fc_ref, ed_ref, gc_ref, c_ref, o_ref, cat_ref):
        blk = pl.program_id(1)

        @pl.when(blk == 0)
        def _():
            diff = ed_ref[0] - c_ref[...]
            rbf = fc_ref[0] * jnp.exp(np.float32(-w) * (diff * diff))
            hi = rbf.astype(jnp.bfloat16)
            lo = (rbf - hi.astype(jnp.float32)).astype(jnp.bfloat16)
            cat_ref[...] = jnp.concatenate([hi, hi, lo, lo], axis=0)

        o_ref[...] = lax.dot_general(
            cat_ref[...], gc_ref[0],
            dimension_numbers=(((0,), (0,)), ((), ())),
            preferred_element_type=jnp.float32)

    return pl.pallas_call(
        body,
        grid=(nt, NBLK),
        in_specs=[
            pl.BlockSpec((1, 1, _TE), lambda i, b: (i, 0, 0)),
            pl.BlockSpec((1, 1, _TE), lambda i, b: (i, 0, 0)),
            pl.BlockSpec((1, 4 * K, F), lambda i, b: (b, 0, 0)),
            pl.BlockSpec((K, 1), lambda i, b: (0, 0)),
        ],
        out_specs=pl.BlockSpec((_TE, F), lambda i, b: (b * nt + i, 0)),
        out_shape=jax.ShapeDtypeStruct((NBLK * e, F), jnp.float32),
        scratch_shapes=[pltpu.VMEM((4 * K, _TE), jnp.bfloat16)],
    )(fcut, expd, gcat, centers)


def _tc_init(zf, mf, qa, qb, w0, b0, w1, b1, w2, b2, wi, bi, wj, bj):
    n = zf.shape[0]
    nt = n // _T

    def body(z_ref, m_ref, a_ref, q_ref, w0_ref, b0_ref, w1_ref, b1_ref,
             w2_ref, b2_ref, wi_ref, bi_ref, wj_ref, bj_ref,
             x_ref, xi_ref, xj_ref):
        pre = (z_ref[...] * w0_ref[0:1, :] + m_ref[...] * w0_ref[1:2, :]
               + a_ref[...] * w0_ref[2:3, :] + q_ref[...] * w0_ref[3:4, :]
               + b0_ref[...])
        x = _act(pre)
        x = _act(_mm(x, w1_ref[...]) + b1_ref[...])
        x = _mm(x, w2_ref[...]) + b2_ref[...]
        xt = _act(x)
        x_ref[...] = x
        xi_ref[...] = _act(_mm(xt, wi_ref[...]) + bi_ref[...])
        xj_ref[...] = _act(_mm(xt, wj_ref[...]) + bj_ref[...])

    col = pl.BlockSpec((_T, 1), lambda i: (i, 0))
    row = lambda r: pl.BlockSpec((r, F), lambda i: (0, 0))
    tile = pl.BlockSpec((_T, F), lambda i: (i, 0))
    return pl.pallas_call(
        body,
        grid=(nt,),
        in_specs=[col, col, col, col,
                  row(4), row(1), row(F), row(1), row(F), row(1),
                  row(F), row(1), row(F), row(1)],
        out_specs=[tile, tile, tile],
        out_shape=[jax.ShapeDtypeStruct((n, F), jnp.float32)] * 3,
    )(zf, mf, qa, qb, w0, b0, w1, b1, w2, b2, wi, bi, wj, bj)


def _wspec(a):
    nd = a.ndim
    return pl.BlockSpec(a.shape, lambda i, _nd=nd: (0,) * _nd)


_TILE = pl.BlockSpec((_T, F), lambda i: (i, 0))
_SCAL = pl.BlockSpec((1, 1), lambda i: (0, 0))


def _tc_block_a(x, xi, parts, wts):
    n = x.shape[0]
    nt = n // _T

    def body(x_ref, xi_ref, p_ref,
             wri1, bri1, wri2, bri2, wm, bm, wra1, bra1, wra2, bra2,
             win, bin_, wjn, bjn,
             xo_ref, xio_ref, xjo_ref):
        m = xi_ref[...] + p_ref[0] + p_ref[1]
        for t in range(NRI):
            m = m + _mm(_act(_mm(_act(m), wri1[t]) + bri1[t]), wri2[t]) + bri2[t]
        x = x_ref[...] + _mm(_act(m), wm[...]) + bm[...]
        for t in range(NRA):
            x = x + _mm(_act(_mm(_act(x), wra1[t]) + bra1[t]), wra2[t]) + bra2[t]
        xo_ref[...] = x
        xt = _act(x)
        xio_ref[...] = _act(_mm(xt, win[...]) + bin_[...])
        xjo_ref[...] = _act(_mm(xt, wjn[...]) + bjn[...])

    pspec = pl.BlockSpec((_NC, _T, F), lambda i: (0, i, 0))
    return pl.pallas_call(
        body,
        grid=(nt,),
        in_specs=[_TILE, _TILE, pspec] + [_wspec(a) for a in wts],
        out_specs=[_TILE, _TILE, _TILE],
        out_shape=[jax.ShapeDtypeStruct((n, F), jnp.float32)] * 3,
    )(x, xi, parts, *wts)


def _tc_block_b(x, outacc, lo2, nh, wts, gate):
    n = x.shape[0]
    nt = n // _T
    inv = float(gate) / float(n * NOUT)

    def body(x_ref, oa_ref, lo_ref, nh_ref,
             wro1, bro1, wro2, bro2, wout, bout, wmix, bmix,
             oao_ref, loo_ref, nho_ref):
        o = x_ref[...]
        for t in range(NRO):
            o = o + _mm(_act(_mm(_act(o), wro1[t]) + bro1[t]), wro2[t]) + bro2[t]
        out = _mm(_act(o), wout[...]) + bout[...]
        oao_ref[...] = oa_ref[...] + out * wmix[...] + bmix[...]
        out2 = out * out
        frac = out2 / (out2 + lo_ref[...] + np.float32(1e-7))
        loo_ref[...] = out2
        part = jnp.sum(frac) * np.float32(inv)

        @pl.when(pl.program_id(0) == 0)
        def _():
            nho_ref[...] = nh_ref[...] + part

        @pl.when(pl.program_id(0) != 0)
        def _():
            nho_ref[...] = nho_ref[...] + part

    return pl.pallas_call(
        body,
        grid=(nt,),
        in_specs=[_TILE, _TILE, _TILE, _SCAL] + [_wspec(a) for a in wts],
        out_specs=[_TILE, _TILE, _SCAL],
        out_shape=[jax.ShapeDtypeStruct((n, F), jnp.float32)] * 2
        + [jax.ShapeDtypeStruct((1, 1), jnp.float32)],
    )(x, outacc, lo2, nh, *wts)


def _tc_final(outp, zcol, scp, shp):
    n = outp.shape[0]
    nt = n // _T

    def body(o_ref, z_ref, sc_ref, sh_ref, r_ref):
        ids = lax.broadcasted_iota(jnp.int32, (_T, F), 1)
        oh = (ids == z_ref[...]).astype(jnp.float32)
        sc = _mm(oh, sc_ref[...])
        sh = _mm(oh, sh_ref[...])
        r_ref[...] = o_ref[...] * sc + sh

    tile = pl.BlockSpec((_T, F), lambda i: (i, 0))
    col = pl.BlockSpec((_T, 1), lambda i: (i, 0))
    full = pl.BlockSpec((F, F), lambda i: (0, 0))
    return pl.pallas_call(
        body,
        grid=(nt,),
        in_specs=[tile, col, full, full],
        out_specs=tile,
        out_shape=jax.ShapeDtypeStruct((n, F), jnp.float32),
    )(outp, zcol, scp, shp)


def kernel(Z, R, idx_i, idx_j, M, QaAlpha, QaBeta, batch_seg, params):
    n = Z.shape[0]
    e = idx_i.shape[0]
    p = params

    ii = idx_i.astype(jnp.int32)
    jj = idx_j.astype(jnp.int32)
    rx = R[:, 0]
    ry = R[:, 1]
    rz = R[:, 2]

    d2 = _sc_d2(rx, ry, rz, ii, jj)

    centers = jnp.linspace(
        np.float32(np.exp(-SR_CUT)), 1.0, K).astype(jnp.float32).reshape(1, K)
    fcut, expd = _tc_edge_prep(d2.reshape(e // _TE, _TE))
    ghi = p['G'].astype(jnp.bfloat16)
    glo = (p['G'] - ghi.astype(jnp.float32)).astype(jnp.bfloat16)
    gcat = jnp.concatenate([ghi, glo, ghi, glo], axis=1)
    gflat = _tc_g(fcut.reshape(e // _TE, 1, _TE),
                  expd.reshape(e // _TE, 1, _TE), gcat, centers.reshape(K, 1))

    zf = Z.astype(jnp.float32).reshape(n, 1)
    x, xi, xj = _tc_init(
        zf, M.reshape(n, 1), QaAlpha.reshape(n, 1), QaBeta.reshape(n, 1),
        p['Wem0'], p['bem0'].reshape(1, F),
        p['Wem1'], p['bem1'].reshape(1, F),
        p['Wem2'], p['bem2'].reshape(1, F),
        p['Wi'][0], p['bi'][0].reshape(1, F),
        p['Wj'][0], p['bj'][0].reshape(1, F))

    woutp = jnp.zeros((NBLK, F, F), jnp.float32).at[:, :, :NOUT].set(p['Wout'])
    boutp = jnp.zeros((NBLK, 1, F), jnp.float32).at[:, 0, :NOUT].set(p['bout'])
    wmixp = jnp.zeros((NBLK, 1, F), jnp.float32).at[:, 0, :NOUT].set(p['Wmix'])
    bmixp = jnp.zeros((1, F), jnp.float32).at[0, :NOUT].set(p['bmix'])

    outacc = jnp.zeros((n, F), jnp.float32)
    lo2 = jnp.zeros((n, F), jnp.float32)
    nh = jnp.zeros((1, 1), jnp.float32)

    for b in range(NBLK):
        parts = _sc_msg(gflat, xj, ii, jj, b)
        nb = (b + 1) % NBLK
        wts_a = (
            p['Wri1'][b], p['bri1'][b].reshape(NRI, 1, F),
            p['Wri2'][b], p['bri2'][b].reshape(NRI, 1, F),
            p['Wm'][b], p['bm'][b].reshape(1, F),
            p['Wra1'][b], p['bra1'][b].reshape(NRA, 1, F),
            p['Wra2'][b], p['bra2'][b].reshape(NRA, 1, F),
            p['Wi'][nb], p['bi'][nb].reshape(1, F),
            p['Wj'][nb], p['bj'][nb].reshape(1, F),
        )
        wts_b = (
            p['Wro1'][b], p['bro1'][b].reshape(NRO, 1, F),
            p['Wro2'][b], p['bro2'][b].reshape(NRO, 1, F),
            woutp[b], boutp[b], wmixp[b], bmixp,
        )
        x, xi, xj = _tc_block_a(x, xi, parts, wts_a)
        outacc, lo2, nh = _tc_block_b(
            x, outacc, lo2, nh, wts_b, gate=1.0 if b > 0 else 0.0)

    nz = p['scales'].shape[1]
    scp = jnp.zeros((F, F), jnp.float32).at[:nz, :NOUT].set(p['scales'].T)
    shp = jnp.zeros((F, F), jnp.float32).at[:nz, :NOUT].set(p['shifts'].T)
    res = _tc_final(outacc, Z.astype(jnp.int32).reshape(n, 1), scp, shp)
    return res[:, :NOUT], nh[0, 0]
